# baseline (speedup 1.0000x reference)
import functools

import jax
import jax.numpy as jnp
from jax import lax
from jax.experimental import pallas as pl
from jax.experimental.pallas import tpu as pltpu
from jax.experimental.pallas import tpu_sc as plsc

NPOINT = [1024, 256]
RADII = [0.2, 0.4]
NSAMPLE = [32, 64]


def _proj_kernel(feat_ref, xyz_ref, wf_ref, wx_ref, b_ref, out_ref):
    u = jnp.dot(feat_ref[0], wf_ref[...], preferred_element_type=jnp.float32)
    u += jnp.dot(xyz_ref[0], wx_ref[...], preferred_element_type=jnp.float32)
    out_ref[0] = u + b_ref[...]


def _project(feat, xyz, wf, wx, b):
    B, N, C = feat.shape
    D = wf.shape[1]
    return pl.pallas_call(
        _proj_kernel,
        grid=(B,),
        in_specs=[
            pl.BlockSpec((1, N, C), lambda b_: (b_, 0, 0)),
            pl.BlockSpec((1, N, 3), lambda b_: (b_, 0, 0)),
            pl.BlockSpec((C, D), lambda b_: (0, 0)),
            pl.BlockSpec((3, D), lambda b_: (0, 0)),
            pl.BlockSpec((1, D), lambda b_: (0, 0)),
        ],
        out_specs=pl.BlockSpec((1, N, D), lambda b_: (b_, 0, 0)),
        out_shape=jax.ShapeDtypeStruct((B, N, D), jnp.float32),
    )(feat, xyz, wf, wx, b.reshape(1, D))


def _thresh_kernel(cxyz_ref, xyz_ref, g_ref, bm_ref, *, k, r2):
    cx = cxyz_ref[0]
    xz = xyz_ref[0]
    dx = cx[:, 0:1] - xz[:, 0][None, :]
    dy = cx[:, 1:2] - xz[:, 1][None, :]
    dz = cx[:, 2:3] - xz[:, 2][None, :]
    d2 = dx * dx + dy * dy + dz * dz
    bits = lax.bitcast_convert_type(d2, jnp.int32)
    v = jnp.zeros((d2.shape[0], 1), jnp.int32)
    for bit in range(30, -1, -1):
        t = v + (1 << bit)
        cnt = jnp.sum((bits < t).astype(jnp.int32), axis=1, keepdims=True)
        v = jnp.where(cnt <= k - 1, t, v)
    tau = lax.bitcast_convert_type(v, jnp.float32)
    thr = jnp.minimum(tau, r2)
    valid = d2 <= thr
    lane = lax.broadcasted_iota(jnp.int32, (1, d2.shape[1]), 1)
    sub = lane % 32
    wlo = jnp.where(sub < 16, 1 << (sub & 15), 0).astype(jnp.float32)
    whi = jnp.where(sub >= 16, 1 << (sub & 15), 0).astype(jnp.float32)
    blo = jnp.where(valid, wlo, 0.0)
    bhi = jnp.where(valid, whi, 0.0)
    plo = jnp.dot(blo, g_ref[...], preferred_element_type=jnp.float32)
    phi = jnp.dot(bhi, g_ref[...], preferred_element_type=jnp.float32)
    bm_ref[0] = plo.astype(jnp.int32) + (phi.astype(jnp.int32) << 16)


def _threshold_bitmask(cxyz, xyz, k, r2, tp):
    B, NP, _ = cxyz.shape
    N = xyz.shape[1]
    nw = N // 32
    g = (jnp.arange(N)[:, None] // 32 == jnp.arange(nw)[None, :]
         ).astype(jnp.float32)
    return pl.pallas_call(
        functools.partial(_thresh_kernel, k=k, r2=r2),
        grid=(B, NP // tp),
        in_specs=[
            pl.BlockSpec((1, tp, 3), lambda b_, p: (b_, p, 0)),
            pl.BlockSpec((1, N, 3), lambda b_, p: (b_, 0, 0)),
            pl.BlockSpec((N, nw), lambda b_, p: (0, 0)),
        ],
        out_specs=pl.BlockSpec((1, tp, nw), lambda b_, p: (b_, p, 0)),
        out_shape=jax.ShapeDtypeStruct((B, NP, nw), jnp.int32),
    )(cxyz, xyz, g)


def _head_kernel(m_ref, cxyz_ref, wx_ref, wf2_ref, wx2_ref, b2_ref, out_ref):
    cx = cxyz_ref[0]
    h = jnp.dot(cx, wx_ref[...], preferred_element_type=jnp.float32)
    f = jnp.maximum(m_ref[0] - h, 0.0)
    u = jnp.dot(f, wf2_ref[...], preferred_element_type=jnp.float32)
    u += jnp.dot(cx, wx2_ref[...], preferred_element_type=jnp.float32)
    out_ref[0] = u + b2_ref[...]


def _head(m, cxyz, wx, wf2, wx2, b2):
    B, NP, D = m.shape
    D2 = wf2.shape[1]
    return pl.pallas_call(
        _head_kernel,
        grid=(B,),
        in_specs=[
            pl.BlockSpec((1, NP, D), lambda b_: (b_, 0, 0)),
            pl.BlockSpec((1, NP, 3), lambda b_: (b_, 0, 0)),
            pl.BlockSpec((3, D), lambda b_: (0, 0)),
            pl.BlockSpec((D, D2), lambda b_: (0, 0)),
            pl.BlockSpec((3, D2), lambda b_: (0, 0)),
            pl.BlockSpec((1, D2), lambda b_: (0, 0)),
        ],
        out_specs=pl.BlockSpec((1, NP, D2), lambda b_: (b_, 0, 0)),
        out_shape=jax.ShapeDtypeStruct((B, NP, D2), jnp.float32),
    )(m, cxyz, wx, wf2, wx2, b2.reshape(1, D2))


def _final_kernel(m_ref, cxyz_ref, wx_ref, wfc_ref, bfc_ref, out_ref):
    h = jnp.dot(cxyz_ref[0], wx_ref[...], preferred_element_type=jnp.float32)
    f = jnp.maximum(m_ref[0] - h, 0.0)
    out_ref[0] = (jnp.dot(f, wfc_ref[...], preferred_element_type=jnp.float32)
                  + bfc_ref[...])


def _final(m, cxyz, wx, wfc, bfc):
    B, NP, D = m.shape
    D2 = wfc.shape[1]
    return pl.pallas_call(
        _final_kernel,
        grid=(B,),
        in_specs=[
            pl.BlockSpec((1, NP, D), lambda b_: (b_, 0, 0)),
            pl.BlockSpec((1, NP, 3), lambda b_: (b_, 0, 0)),
            pl.BlockSpec((3, D), lambda b_: (0, 0)),
            pl.BlockSpec((D, D2), lambda b_: (0, 0)),
            pl.BlockSpec((1, D2), lambda b_: (0, 0)),
        ],
        out_specs=pl.BlockSpec((1, NP, D2), lambda b_: (b_, 0, 0)),
        out_shape=jax.ShapeDtypeStruct((B, NP, D2), jnp.float32),
    )(m, cxyz, wx, wfc, bfc.reshape(1, D2))


def _sc_gather_max(bm, u_flat, *, B, N, NP, k, D, gb):
    NW = 32
    rpw = (B * NP) // NW
    kpad = k + 16
    nwords = N // 32
    ngrp = max(nwords // 16, 1)
    ndc = D // 16
    nchunk = rpw // gb
    mesh = plsc.VectorSubcoreMesh(core_axis_name="c", subcore_axis_name="s")

    @functools.partial(
        pl.kernel,
        out_type=jax.ShapeDtypeStruct((B * NP, D), jnp.float32),
        mesh=mesh,
        scratch_types=[
            pltpu.VMEM((rpw, nwords), jnp.int32),
            pltpu.VMEM((kpad,), jnp.int32),
            pltpu.VMEM((rpw * k,), jnp.int32),
            pltpu.VMEM((gb * k, D), jnp.float32),
            pltpu.VMEM((gb * k, D), jnp.float32),
            pltpu.VMEM((rpw, D), jnp.float32),
            pltpu.SMEM((264,), jnp.int32),
            pltpu.SemaphoreType.DMA,
            pltpu.SemaphoreType.DMA,
        ],
    )
    def sc_kernel(bm_hbm, u_hbm, out_hbm,
                  bmv, idxbig, idxall, rowsa, rowsb, macc, sm, sema, semb):
        wid = lax.axis_index("s") * 2 + lax.axis_index("c")
        row0 = wid * rpw
        bb = row0 // NP
        p0 = row0 - bb * NP
        gbase = bb * N
        pltpu.sync_copy(bm_hbm.at[pl.ds(row0, rpw)], bmv)
        for vv in range(256):
            sm[8 + vv] = bin(vv).count("1")

        def row_scan(i, _):
            sm[0] = 0
            csplat = jnp.full((16,), gbase + p0 + i, jnp.int32)
            for q in range(kpad // 16):
                idxbig[pl.ds(q * 16, 16)] = csplat

            def grp(g, _2):
                bv = bmv[i, pl.ds(g * 16, 16)]
                for s in range(16):
                    w = bv[s]

                    @pl.when(w != 0)
                    def _word():
                        pc = (sm[8 + (w & 255)]
                              + sm[8 + ((w >> 8) & 255)]
                              + sm[8 + ((w >> 16) & 255)]
                              + sm[8 + ((w >> 24) & 255)])

                        @pl.loop(0, pc, init_carry=w)
                        def _bit(t, w2):
                            low = w2 & (-w2)
                            fb = lax.bitcast_convert_type(
                                lax.convert_element_type(low, jnp.float32),
                                jnp.int32)
                            ctz = ((fb >> 23) & 255) - 127
                            c = sm[0]
                            idxbig[pl.ds(c, 16)] = jnp.full(
                                (16,), gbase + (g * 16 + s) * 32 + ctz,
                                jnp.int32)
                            sm[0] = jnp.minimum(c + 1, k)
                            return w2 & (w2 - 1)
                return 0

            lax.fori_loop(0, ngrp, grp, 0)
            for q in range(k // 16):
                idxall[pl.ds(i * k + q * 16, 16)] = idxbig[pl.ds(q * 16, 16)]
            return 0

        def decode_chunk(c):
            lax.fori_loop(c * gb, (c + 1) * gb, row_scan, 0)

        def fire(c, buf, sem):
            pltpu.make_async_copy(
                u_hbm.at[idxall.at[pl.ds(c * gb * k, gb * k)]],
                buf, sem).start()

        def drain(c, buf, sem):
            pltpu.make_async_copy(
                u_hbm.at[idxall.at[pl.ds(c * gb * k, gb * k)]],
                buf, sem).wait()

        def reduce_chunk(c, buf):
            for rr in range(gb):
                for half in range(ndc // 16):
                    h0 = half * 16

                    def rstep(r, acc):
                        return tuple(
                            jnp.maximum(acc[dc],
                                        buf[rr * k + r,
                                            pl.ds((h0 + dc) * 16, 16)])
                            for dc in range(16))

                    acc0 = tuple(buf[rr * k, pl.ds((h0 + dc) * 16, 16)]
                                 for dc in range(16))
                    accf = lax.fori_loop(1, k, rstep, acc0, unroll=4)
                    for dc in range(16):
                        macc[c * gb + rr, pl.ds((h0 + dc) * 16, 16)] = accf[dc]

        decode_chunk(0)
        fire(0, rowsa, sema)
        decode_chunk(1)
        fire(1, rowsb, semb)

        def pair(j, _):
            c0 = j * 2

            @pl.when(c0 + 2 < nchunk)
            def _():
                decode_chunk(c0 + 2)

            drain(c0, rowsa, sema)
            reduce_chunk(c0, rowsa)

            @pl.when(c0 + 2 < nchunk)
            def _():
                fire(c0 + 2, rowsa, sema)

            @pl.when(c0 + 3 < nchunk)
            def _():
                decode_chunk(c0 + 3)

            drain(c0 + 1, rowsb, semb)
            reduce_chunk(c0 + 1, rowsb)

            @pl.when(c0 + 3 < nchunk)
            def _():
                fire(c0 + 3, rowsb, semb)

            return 0

        lax.fori_loop(0, nchunk // 2, pair, 0)
        pltpu.sync_copy(macc, out_hbm.at[pl.ds(row0, rpw)])

    return sc_kernel(bm, u_flat)


def kernel(feat, xyz, obj_masks, downsample_mask, W0, b0, W1, b1, Wfc, bfc):
    B, N, C0 = feat.shape
    C1 = W0.shape[1]
    D1 = W1.shape[1]
    NP0, NP1 = NPOINT
    cxyz0 = xyz[:, :NP0]

    u0 = _project(feat, xyz, W0[:C0], W0[C0:], b0)
    bm0 = _threshold_bitmask(cxyz0, xyz, NSAMPLE[0], RADII[0] ** 2, tp=256)
    m0 = _sc_gather_max(
        bm0.reshape(B * NP0, N // 32), u0.reshape(B * N, C1),
        B=B, N=N, NP=NP0, k=NSAMPLE[0], D=C1, gb=4).reshape(B, NP0, C1)

    u1 = _head(m0, cxyz0, W0[C0:], W1[:C1], W1[C1:], b1)
    cxyz1 = xyz[:, :NP1]
    bm1 = _threshold_bitmask(cxyz1, cxyz0, NSAMPLE[1], RADII[1] ** 2, tp=256)
    m1 = _sc_gather_max(
        bm1.reshape(B * NP1, NP0 // 32), u1.reshape(B * NP0, D1),
        B=B, N=NP0, NP=NP1, k=NSAMPLE[1], D=D1, gb=1).reshape(B, NP1, D1)

    f = _final(m1, cxyz1, W1[C1:], Wfc, bfc)
    return (f, obj_masks[:, :NP1])


# speedup vs baseline: 1.0008x; 1.0008x over previous
import functools

import jax
import jax.numpy as jnp
from jax import lax
from jax.experimental import pallas as pl
from jax.experimental.pallas import tpu as pltpu
from jax.experimental.pallas import tpu_sc as plsc

NPOINT = [1024, 256]
RADII = [0.2, 0.4]
NSAMPLE = [32, 64]


def _proj_kernel(feat_ref, xyz_ref, wf_ref, wx_ref, b_ref, out_ref):
    u = jnp.dot(feat_ref[0], wf_ref[...], preferred_element_type=jnp.float32)
    u += jnp.dot(xyz_ref[0], wx_ref[...], preferred_element_type=jnp.float32)
    out_ref[0] = u + b_ref[...]


def _project(feat, xyz, wf, wx, b):
    B, N, C = feat.shape
    D = wf.shape[1]
    return pl.pallas_call(
        _proj_kernel,
        grid=(B,),
        in_specs=[
            pl.BlockSpec((1, N, C), lambda b_: (b_, 0, 0)),
            pl.BlockSpec((1, N, 3), lambda b_: (b_, 0, 0)),
            pl.BlockSpec((C, D), lambda b_: (0, 0)),
            pl.BlockSpec((3, D), lambda b_: (0, 0)),
            pl.BlockSpec((1, D), lambda b_: (0, 0)),
        ],
        out_specs=pl.BlockSpec((1, N, D), lambda b_: (b_, 0, 0)),
        out_shape=jax.ShapeDtypeStruct((B, N, D), jnp.float32),
    )(feat, xyz, wf, wx, b.reshape(1, D))


def _thresh_kernel(cxyz_ref, xyz_ref, g_ref, bm_ref, *, k, r2):
    cx = cxyz_ref[0]
    xz = xyz_ref[0]
    dx = cx[:, 0:1] - xz[:, 0][None, :]
    dy = cx[:, 1:2] - xz[:, 1][None, :]
    dz = cx[:, 2:3] - xz[:, 2][None, :]
    d2 = dx * dx + dy * dy + dz * dz
    bits = lax.bitcast_convert_type(d2, jnp.int32)
    v = jnp.zeros((d2.shape[0], 1), jnp.int32)
    for bit in range(30, -1, -1):
        t = v + (1 << bit)
        cnt = jnp.sum((bits < t).astype(jnp.int32), axis=1, keepdims=True)
        v = jnp.where(cnt <= k - 1, t, v)
    tau = lax.bitcast_convert_type(v, jnp.float32)
    thr = jnp.minimum(tau, r2)
    valid = d2 <= thr
    lane = lax.broadcasted_iota(jnp.int32, (1, d2.shape[1]), 1)
    sub = lane % 32
    wlo = jnp.where(sub < 16, 1 << (sub & 15), 0).astype(jnp.float32)
    whi = jnp.where(sub >= 16, 1 << (sub & 15), 0).astype(jnp.float32)
    blo = jnp.where(valid, wlo, 0.0)
    bhi = jnp.where(valid, whi, 0.0)
    plo = jnp.dot(blo, g_ref[...], preferred_element_type=jnp.float32)
    phi = jnp.dot(bhi, g_ref[...], preferred_element_type=jnp.float32)
    bm_ref[0] = plo.astype(jnp.int32) + (phi.astype(jnp.int32) << 16)


def _threshold_bitmask(cxyz, xyz, k, r2, tp):
    B, NP, _ = cxyz.shape
    N = xyz.shape[1]
    nw = N // 32
    g = (jnp.arange(N)[:, None] // 32 == jnp.arange(nw)[None, :]
         ).astype(jnp.float32)
    return pl.pallas_call(
        functools.partial(_thresh_kernel, k=k, r2=r2),
        grid=(B, NP // tp),
        in_specs=[
            pl.BlockSpec((1, tp, 3), lambda b_, p: (b_, p, 0)),
            pl.BlockSpec((1, N, 3), lambda b_, p: (b_, 0, 0)),
            pl.BlockSpec((N, nw), lambda b_, p: (0, 0)),
        ],
        out_specs=pl.BlockSpec((1, tp, nw), lambda b_, p: (b_, p, 0)),
        out_shape=jax.ShapeDtypeStruct((B, NP, nw), jnp.int32),
    )(cxyz, xyz, g)


def _head_kernel(m_ref, cxyz_ref, wx_ref, wf2_ref, wx2_ref, b2_ref, out_ref):
    cx = cxyz_ref[0]
    h = jnp.dot(cx, wx_ref[...], preferred_element_type=jnp.float32)
    f = jnp.maximum(m_ref[0] - h, 0.0)
    u = jnp.dot(f, wf2_ref[...], preferred_element_type=jnp.float32)
    u += jnp.dot(cx, wx2_ref[...], preferred_element_type=jnp.float32)
    out_ref[0] = u + b2_ref[...]


def _head(m, cxyz, wx, wf2, wx2, b2):
    B, NP, D = m.shape
    D2 = wf2.shape[1]
    return pl.pallas_call(
        _head_kernel,
        grid=(B,),
        in_specs=[
            pl.BlockSpec((1, NP, D), lambda b_: (b_, 0, 0)),
            pl.BlockSpec((1, NP, 3), lambda b_: (b_, 0, 0)),
            pl.BlockSpec((3, D), lambda b_: (0, 0)),
            pl.BlockSpec((D, D2), lambda b_: (0, 0)),
            pl.BlockSpec((3, D2), lambda b_: (0, 0)),
            pl.BlockSpec((1, D2), lambda b_: (0, 0)),
        ],
        out_specs=pl.BlockSpec((1, NP, D2), lambda b_: (b_, 0, 0)),
        out_shape=jax.ShapeDtypeStruct((B, NP, D2), jnp.float32),
    )(m, cxyz, wx, wf2, wx2, b2.reshape(1, D2))


def _final_kernel(m_ref, cxyz_ref, wx_ref, wfc_ref, bfc_ref, out_ref):
    h = jnp.dot(cxyz_ref[0], wx_ref[...], preferred_element_type=jnp.float32)
    f = jnp.maximum(m_ref[0] - h, 0.0)
    out_ref[0] = (jnp.dot(f, wfc_ref[...], preferred_element_type=jnp.float32)
                  + bfc_ref[...])


def _final(m, cxyz, wx, wfc, bfc):
    B, NP, D = m.shape
    D2 = wfc.shape[1]
    return pl.pallas_call(
        _final_kernel,
        grid=(B,),
        in_specs=[
            pl.BlockSpec((1, NP, D), lambda b_: (b_, 0, 0)),
            pl.BlockSpec((1, NP, 3), lambda b_: (b_, 0, 0)),
            pl.BlockSpec((3, D), lambda b_: (0, 0)),
            pl.BlockSpec((D, D2), lambda b_: (0, 0)),
            pl.BlockSpec((1, D2), lambda b_: (0, 0)),
        ],
        out_specs=pl.BlockSpec((1, NP, D2), lambda b_: (b_, 0, 0)),
        out_shape=jax.ShapeDtypeStruct((B, NP, D2), jnp.float32),
    )(m, cxyz, wx, wfc, bfc.reshape(1, D2))


def _sc_gather_max(bm, u_flat, *, B, N, NP, k, D, gb):
    NW = 32
    rpw = (B * NP) // NW
    kpad = k + 16
    nwords = N // 32
    ngrp = max(nwords // 16, 1)
    ndc = D // 16
    nchunk = rpw // gb
    mesh = plsc.VectorSubcoreMesh(core_axis_name="c", subcore_axis_name="s")

    @functools.partial(
        pl.kernel,
        out_type=jax.ShapeDtypeStruct((B * NP, D), jnp.float32),
        mesh=mesh,
        scratch_types=[
            pltpu.VMEM((rpw, nwords), jnp.int32),
            pltpu.VMEM((kpad,), jnp.int32),
            pltpu.VMEM((rpw * k,), jnp.int32),
            pltpu.VMEM((gb * k, D), jnp.float32),
            pltpu.VMEM((gb * k, D), jnp.float32),
            pltpu.VMEM((rpw, D), jnp.float32),
            pltpu.SMEM((264,), jnp.int32),
            pltpu.SemaphoreType.DMA,
            pltpu.SemaphoreType.DMA,
        ],
    )
    def sc_kernel(bm_hbm, u_hbm, out_hbm,
                  bmv, idxbig, idxall, rowsa, rowsb, macc, sm, sema, semb):
        wid = lax.axis_index("s") * 2 + lax.axis_index("c")
        row0 = wid * rpw
        bb = row0 // NP
        p0 = row0 - bb * NP
        gbase = bb * N
        pltpu.sync_copy(bm_hbm.at[pl.ds(row0, rpw)], bmv)
        for vv in range(256):
            sm[8 + vv] = bin(vv).count("1")

        def row_scan(i, _):
            sm[0] = 0
            csplat = jnp.full((16,), gbase + p0 + i, jnp.int32)
            for q in range(kpad // 16):
                idxbig[pl.ds(q * 16, 16)] = csplat

            def grp(g, _2):
                bv = bmv[i, pl.ds(g * 16, 16)]
                for s in range(16):
                    w = bv[s]

                    @pl.when(w != 0)
                    def _word():
                        pc = (sm[8 + (w & 255)]
                              + sm[8 + ((w >> 8) & 255)]
                              + sm[8 + ((w >> 16) & 255)]
                              + sm[8 + ((w >> 24) & 255)])

                        @pl.loop(0, pc, init_carry=w)
                        def _bit(t, w2):
                            low = w2 & (-w2)
                            fb = lax.bitcast_convert_type(
                                lax.convert_element_type(low, jnp.float32),
                                jnp.int32)
                            ctz = ((fb >> 23) & 255) - 127
                            c = sm[0]
                            idxbig[pl.ds(c, 16)] = jnp.full(
                                (16,), gbase + (g * 16 + s) * 32 + ctz,
                                jnp.int32)
                            sm[0] = jnp.minimum(c + 1, k)
                            return w2 & (w2 - 1)
                return 0

            lax.fori_loop(0, ngrp, grp, 0)
            for q in range(k // 16):
                idxall[pl.ds(i * k + q * 16, 16)] = idxbig[pl.ds(q * 16, 16)]
            return 0

        def decode_chunk(c):
            lax.fori_loop(c * gb, (c + 1) * gb, row_scan, 0)

        def fire(c, buf, sem):
            pltpu.make_async_copy(
                u_hbm.at[idxall.at[pl.ds(c * gb * k, gb * k)]],
                buf, sem).start()

        def drain(c, buf, sem):
            pltpu.make_async_copy(
                u_hbm.at[idxall.at[pl.ds(c * gb * k, gb * k)]],
                buf, sem).wait()

        def reduce_chunk(c, buf):
            for rr in range(gb):
                for half in range(ndc // 16):
                    h0 = half * 16

                    def rstep(r, acc):
                        return tuple(
                            jnp.maximum(acc[dc],
                                        buf[rr * k + r,
                                            pl.ds((h0 + dc) * 16, 16)])
                            for dc in range(16))

                    acc0 = tuple(buf[rr * k, pl.ds((h0 + dc) * 16, 16)]
                                 for dc in range(16))
                    accf = lax.fori_loop(1, k, rstep, acc0, unroll=4)
                    for dc in range(16):
                        macc[c * gb + rr, pl.ds((h0 + dc) * 16, 16)] = accf[dc]

        decode_chunk(0)
        fire(0, rowsa, sema)
        decode_chunk(1)
        fire(1, rowsb, semb)

        def pair(j, _):
            c0 = j * 2

            @pl.when(c0 + 2 < nchunk)
            def _():
                decode_chunk(c0 + 2)

            drain(c0, rowsa, sema)
            reduce_chunk(c0, rowsa)

            @pl.when(c0 + 2 < nchunk)
            def _():
                fire(c0 + 2, rowsa, sema)

            @pl.when(c0 + 3 < nchunk)
            def _():
                decode_chunk(c0 + 3)

            drain(c0 + 1, rowsb, semb)
            reduce_chunk(c0 + 1, rowsb)

            @pl.when(c0 + 3 < nchunk)
            def _():
                fire(c0 + 3, rowsb, semb)

            return 0

        lax.fori_loop(0, nchunk // 2, pair, 0)
        pltpu.sync_copy(macc, out_hbm.at[pl.ds(row0, rpw)])

    return sc_kernel(bm, u_flat)


def kernel(feat, xyz, obj_masks, downsample_mask, W0, b0, W1, b1, Wfc, bfc):
    B, N, C0 = feat.shape
    C1 = W0.shape[1]
    D1 = W1.shape[1]
    NP0, NP1 = NPOINT
    cxyz0 = xyz[:, :NP0]

    u0 = _project(feat, xyz, W0[:C0], W0[C0:], b0)
    bm0 = _threshold_bitmask(cxyz0, xyz, NSAMPLE[0], RADII[0] ** 2, tp=256)
    cxyz1 = xyz[:, :NP1]
    bm1 = _threshold_bitmask(cxyz1, cxyz0, NSAMPLE[1], RADII[1] ** 2, tp=256)
    m0 = _sc_gather_max(
        bm0.reshape(B * NP0, N // 32), u0.reshape(B * N, C1),
        B=B, N=N, NP=NP0, k=NSAMPLE[0], D=C1, gb=4).reshape(B, NP0, C1)

    u1 = _head(m0, cxyz0, W0[C0:], W1[:C1], W1[C1:], b1)
    m1 = _sc_gather_max(
        bm1.reshape(B * NP1, NP0 // 32), u1.reshape(B * NP0, D1),
        B=B, N=NP0, NP=NP1, k=NSAMPLE[1], D=D1, gb=1).reshape(B, NP1, D1)

    f = _final(m1, cxyz1, W1[C1:], Wfc, bfc)
    return (f, obj_masks[:, :NP1])
